# baseline (speedup 1.0000x reference)
import jax
import jax.numpy as jnp
from jax import lax
from jax.experimental import pallas as pl
from jax.experimental.pallas import tpu as pltpu

_H = 32
_W = 32
_E = 768
_BINS = 16
_IMG = 512
_PATCH = 16
_T = _H * _W


_BB = 4


def _body(pt_ref, d_ref, row_ref, col_ref, demb_ref, out_ref, rc_ref):
    b = pl.program_id(0)

    @pl.when(b == 0)
    def _():
        row = row_ref[...]
        col = col_ref[...]
        rc = row[:, None, :] + col[None, :, :]
        rc_ref[...] = rc.reshape(_T, _E)

    a0 = lax.broadcasted_iota(jnp.int32, (_H, _IMG), 0)
    a1 = lax.broadcasted_iota(jnp.int32, (_H, _IMG), 1)
    p1 = jnp.where(a1 // _PATCH == a0, 1.0 / _PATCH, 0.0)
    b0 = lax.broadcasted_iota(jnp.int32, (_IMG, _W), 0)
    b1 = lax.broadcasted_iota(jnp.int32, (_IMG, _W), 1)
    p2 = jnp.where(b0 // _PATCH == b1, 1.0 / _PATCH, 0.0)

    t0 = lax.broadcasted_iota(jnp.int32, (_T, _H), 0)
    t1 = lax.broadcasted_iota(jnp.int32, (_T, _H), 1)
    onehot_r = jnp.where(t0 // _W == t1, 1.0, 0.0)
    onehot_c = jnp.where(t0 % _W == t1, 1.0, 0.0)
    k = lax.broadcasted_iota(jnp.int32, (_T, _BINS), 1)

    for j in range(_BB):
        d = d_ref[j]
        pooled = jnp.dot(jnp.dot(p1, d), p2)
        dpos = jnp.sqrt(jnp.clip(pooled, 0.0, 1.0)) * (_BINS - 1)

        rowsel = jnp.dot(onehot_r, dpos)
        dpos_col = jnp.sum(rowsel * onehot_c, axis=1, keepdims=True)

        lo_f = jnp.floor(dpos_col)
        alpha = dpos_col - lo_f
        lo = lo_f.astype(jnp.int32)
        hi = jnp.minimum(lo + 1, _BINS - 1)
        w = jnp.where(k == lo, 1.0 - alpha, 0.0) + jnp.where(k == hi, alpha, 0.0)
        depth_pe = jnp.dot(w, demb_ref[...])

        out_ref[j] = pt_ref[j] + rc_ref[...] + depth_pe


def kernel(patch_tokens, depth_ch, row_emb, col_emb, depth_emb):
    bsz = patch_tokens.shape[0]
    return pl.pallas_call(
        _body,
        grid=(bsz // _BB,),
        in_specs=[
            pl.BlockSpec((_BB, _T, _E), lambda b: (b, 0, 0)),
            pl.BlockSpec((_BB, _IMG, _IMG), lambda b: (b, 0, 0)),
            pl.BlockSpec((_H, _E), lambda b: (0, 0)),
            pl.BlockSpec((_W, _E), lambda b: (0, 0)),
            pl.BlockSpec((_BINS, _E), lambda b: (0, 0)),
        ],
        out_specs=pl.BlockSpec((_BB, _T, _E), lambda b: (b, 0, 0)),
        out_shape=jax.ShapeDtypeStruct((bsz, _T, _E), jnp.float32),
        scratch_shapes=[pltpu.VMEM((_T, _E), jnp.float32)],
        compiler_params=pltpu.CompilerParams(
            dimension_semantics=("arbitrary",),
            vmem_limit_bytes=100 * 1024 * 1024,
        ),
    )(patch_tokens, depth_ch.reshape(bsz, _IMG, _IMG), row_emb, col_emb, depth_emb)
